# baseline (speedup 1.0000x reference)
import functools

import jax
import jax.numpy as jnp
from jax import lax
from jax.experimental import pallas as pl
from jax.experimental.pallas import tpu as pltpu
from jax.experimental.pallas import tpu_sc as plsc

N_PAD = 10112
LANES = 16
CHUNK = 128
NTILES = 32
NBLOCKS = N_PAD // CHUNK
NBUF = 3


def _mesh():
    return plsc.VectorSubcoreMesh(core_axis_name="c", subcore_axis_name="s",
                                  num_cores=2, num_subcores=16)


def _tile_blocks(s):
    return [(k, s + k * 16) for k in range((NBLOCKS + 15) // 16)]


def _zero_vmem_block(buf_ref, d):
    def zbody(i, _):
        r = i // (d // LANES)
        col = (i % (d // LANES)) * LANES
        buf_ref[r, pl.ds(col, LANES)] = jnp.zeros((LANES,), jnp.float32)
        return 0
    lax.fori_loop(0, CHUNK * (d // LANES), zbody, 0)


def _sc_degree(dst1d, ones):
    g_per_tile = dst1d.shape[0] // (NTILES * CHUNK)
    d = ones.shape[1]

    @functools.partial(
        pl.kernel,
        out_type=jax.ShapeDtypeStruct((2, N_PAD, d), jnp.float32),
        mesh=_mesh(),
        scratch_types=[
            pltpu.VMEM((NBUF, CHUNK), jnp.int32),
            pltpu.VMEM((CHUNK, d), jnp.float32),
            pltpu.VMEM_SHARED((N_PAD, d), jnp.float32),
            [pltpu.SemaphoreType.DMA] * NBUF,
        ],
    )
    def k(dst_hbm, ones_hbm, out_hbm, ib_v, ones_v, deg_sh, isems):
        c = lax.axis_index("c")
        s = lax.axis_index("s")
        wid = c * 16 + s
        base = wid * g_per_tile

        _zero_vmem_block(ones_v, d)
        for _, blk in _tile_blocks(s):
            @pl.when(blk < NBLOCKS)
            def _():
                pltpu.sync_copy(ones_v, deg_sh.at[pl.ds(blk * CHUNK, CHUNK)])
        plsc.subcore_barrier()
        pltpu.sync_copy(ones_hbm, ones_v)

        def idx_load(g, b):
            pltpu.async_copy(dst_hbm.at[pl.ds((base + g) * CHUNK, CHUNK)],
                             ib_v.at[b], isems[b])

        for j in range(NBUF):
            idx_load(j, j)

        def step(t, _):
            for b in range(NBUF):
                g = t * NBUF + b
                pltpu.make_async_copy(dst_hbm.at[pl.ds((base + g) * CHUNK, CHUNK)],
                                      ib_v.at[b], isems[b]).wait()
                pltpu.sync_copy(ones_v, deg_sh.at[ib_v.at[b]], add=True)

                @pl.when(g + NBUF < g_per_tile)
                def _():
                    idx_load(g + NBUF, b)
            return 0
        lax.fori_loop(0, g_per_tile // NBUF, step, 0)

        plsc.subcore_barrier()
        for _, blk in _tile_blocks(s):
            @pl.when(blk < NBLOCKS)
            def _():
                pltpu.sync_copy(deg_sh.at[pl.ds(blk * CHUNK, CHUNK)],
                                out_hbm.at[c, pl.ds(blk * CHUNK, CHUNK)])

    return k(dst1d, ones)


NBUF_A = 6
GPT_A = 96


def _sc_gather(y, src2d):
    g_per_tile = src2d.shape[0] // NTILES
    assert g_per_tile % NBUF_A == 0 and g_per_tile % 8 == 0
    d = y.shape[1]
    n_rows = src2d.shape[0] * CHUNK

    @functools.partial(
        pl.kernel,
        out_type=jax.ShapeDtypeStruct((n_rows, d), jnp.float32),
        mesh=_mesh(),
        scratch_types=[
            pltpu.VMEM((g_per_tile, CHUNK), jnp.int32),
            pltpu.VMEM((NBUF_A, CHUNK, d), jnp.float32),
            [pltpu.SemaphoreType.DMA] * NBUF_A,
            [pltpu.SemaphoreType.DMA] * NBUF_A,
        ],
    )
    def k(y_hbm, src_hbm, out_hbm, src_v, rows_v, gsems, wsems):
        c = lax.axis_index("c")
        s = lax.axis_index("s")
        wid = c * 16 + s
        base = wid * g_per_tile

        pltpu.sync_copy(src_hbm.at[pl.ds(base, g_per_tile)], src_v)

        def gather(g, b):
            pltpu.async_copy(y_hbm.at[src_v.at[g]], rows_v.at[b], gsems[b])

        def gather_wait(g, b):
            pltpu.make_async_copy(y_hbm.at[src_v.at[g]], rows_v.at[b],
                                  gsems[b]).wait()

        def write(g, b):
            pltpu.async_copy(rows_v.at[b],
                             out_hbm.at[pl.ds((base + g) * CHUNK, CHUNK)],
                             wsems[b])

        def write_wait(g, b):
            pltpu.make_async_copy(rows_v.at[b],
                                  out_hbm.at[pl.ds((base + g) * CHUNK, CHUNK)],
                                  wsems[b]).wait()

        for j in range(NBUF_A - 1):
            gather(j, j)
        for j in range(NBUF_A):
            gather_wait(j, j)
            write(j, j)
            if j >= 1:
                write_wait(j - 1, j - 1)
            gather(j + NBUF_A - 1, (j + NBUF_A - 1) % NBUF_A)

        def step(t, _):
            for j in range(NBUF_A):
                g = t * NBUF_A + j
                b = j
                bn = (j + NBUF_A - 1) % NBUF_A
                gather_wait(g, b)
                write(g, b)
                @pl.when(g + NBUF_A - 1 < g_per_tile)
                def _():
                    write_wait(g - 1, bn)
                    gather(g + NBUF_A - 1, bn)
            return 0
        lax.fori_loop(1, g_per_tile // NBUF_A, step, 0)
        for j in range(NBUF_A):
            g_last = g_per_tile - NBUF_A + j
            write_wait(g_last, g_last % NBUF_A)

    return k(y, src2d)


def _sc_scatter(msgs, dst1d):
    g_per_tile = dst1d.shape[0] // (NTILES * CHUNK)
    assert g_per_tile % NBUF == 0
    d = msgs.shape[1]

    @functools.partial(
        pl.kernel,
        out_type=jax.ShapeDtypeStruct((2, N_PAD, d), jnp.float32),
        mesh=_mesh(),
        scratch_types=[
            pltpu.VMEM((NBUF, CHUNK), jnp.int32),
            pltpu.VMEM((NBUF, CHUNK, d), jnp.float32),
            pltpu.VMEM_SHARED((N_PAD, d), jnp.float32),
            [pltpu.SemaphoreType.DMA] * NBUF,
            [pltpu.SemaphoreType.DMA] * NBUF,
        ],
    )
    def k(msgs_hbm, dst_hbm, out_hbm, ib_v, rows_v, agg_sh, lsems, isems):
        c = lax.axis_index("c")
        s = lax.axis_index("s")
        wid = c * 16 + s
        base = wid * g_per_tile

        _zero_vmem_block(rows_v.at[0], d)
        for _, blk in _tile_blocks(s):
            @pl.when(blk < NBLOCKS)
            def _():
                pltpu.sync_copy(rows_v.at[0], agg_sh.at[pl.ds(blk * CHUNK, CHUNK)])
        plsc.subcore_barrier()

        def load(g, b):
            off = (base + g) * CHUNK
            pltpu.async_copy(dst_hbm.at[pl.ds(off, CHUNK)], ib_v.at[b], isems[b])
            pltpu.async_copy(msgs_hbm.at[pl.ds(off, CHUNK)], rows_v.at[b],
                             lsems[b])

        def load_wait(g, b):
            off = (base + g) * CHUNK
            pltpu.make_async_copy(dst_hbm.at[pl.ds(off, CHUNK)], ib_v.at[b],
                                  isems[b]).wait()
            pltpu.make_async_copy(msgs_hbm.at[pl.ds(off, CHUNK)], rows_v.at[b],
                                  lsems[b]).wait()

        for j in range(NBUF):
            load(j, j)

        def step(t, _):
            for b in range(NBUF):
                g = t * NBUF + b
                load_wait(g, b)
                pltpu.sync_copy(rows_v.at[b], agg_sh.at[ib_v.at[b]], add=True)

                @pl.when(g + NBUF < g_per_tile)
                def _():
                    load(g + NBUF, b)
            return 0
        lax.fori_loop(0, g_per_tile // NBUF, step, 0)

        plsc.subcore_barrier()
        for _, blk in _tile_blocks(s):
            @pl.when(blk < NBLOCKS)
            def _():
                pltpu.sync_copy(agg_sh.at[pl.ds(blk * CHUNK, CHUNK)],
                                out_hbm.at[c, pl.ds(blk * CHUNK, CHUNK)])

    return k(msgs, dst1d)


def _sc_edge_pass(y, src2d, dst1d):
    msgs = _sc_gather(y, src2d)
    return _sc_scatter(msgs, dst1d)


def _dinv_block(deg_ref):
    deg = deg_ref[0, :, 0:1] + deg_ref[1, :, 0:1]
    return jnp.where(deg > 0, lax.rsqrt(deg), 0.0)


def _tc_in(x, deg_p, W):
    n, d = x.shape

    def body(x_ref, deg_ref, w_ref, y_ref):
        dinv = _dinv_block(deg_ref)
        y_ref[...] = dinv * jnp.dot(x_ref[...], w_ref[...],
                                    preferred_element_type=jnp.float32)

    return pl.pallas_call(
        body,
        grid=(n // 128,),
        in_specs=[
            pl.BlockSpec((128, d), lambda i: (i, 0)),
            pl.BlockSpec((2, 128, 128), lambda i: (0, i, 0)),
            pl.BlockSpec((d, d), lambda i: (0, 0)),
        ],
        out_specs=pl.BlockSpec((128, d), lambda i: (i, 0)),
        out_shape=jax.ShapeDtypeStruct((n, d), jnp.float32),
    )(x, deg_p, W)


def _tc_mid(agg_p, deg_p, b, W):
    n, d = agg_p.shape[1], agg_p.shape[2]

    def body(a_ref, deg_ref, b_ref, w_ref, y_ref):
        dinv = _dinv_block(deg_ref)
        x2 = jnp.maximum(dinv * (a_ref[0] + a_ref[1]) + b_ref[...], 0.0)
        y_ref[...] = dinv * jnp.dot(x2, w_ref[...], preferred_element_type=jnp.float32)

    return pl.pallas_call(
        body,
        grid=(n // 128,),
        in_specs=[
            pl.BlockSpec((2, 128, d), lambda i: (0, i, 0)),
            pl.BlockSpec((2, 128, 128), lambda i: (0, i, 0)),
            pl.BlockSpec((1, d), lambda i: (0, 0)),
            pl.BlockSpec((d, d), lambda i: (0, 0)),
        ],
        out_specs=pl.BlockSpec((128, d), lambda i: (i, 0)),
        out_shape=jax.ShapeDtypeStruct((n, d), jnp.float32),
    )(agg_p, deg_p, b, W)


def _tc_out(agg_p, deg_p, b):
    n, d = agg_p.shape[1], agg_p.shape[2]

    def body(a_ref, deg_ref, b_ref, o_ref):
        dinv = _dinv_block(deg_ref)
        o_ref[...] = jnp.maximum(dinv * (a_ref[0] + a_ref[1]) + b_ref[...], 0.0)

    return pl.pallas_call(
        body,
        grid=(n // 128,),
        in_specs=[
            pl.BlockSpec((2, 128, d), lambda i: (0, i, 0)),
            pl.BlockSpec((2, 128, 128), lambda i: (0, i, 0)),
            pl.BlockSpec((1, d), lambda i: (0, 0)),
        ],
        out_specs=pl.BlockSpec((128, d), lambda i: (i, 0)),
        out_shape=jax.ShapeDtypeStruct((n, d), jnp.float32),
    )(agg_p, deg_p, b)


def kernel(edge_index, emb, W1, b1, W2, b2):
    src, dst = edge_index[0], edge_index[1]
    e = src.shape[0]
    n, d = emb.shape

    unit = NTILES * CHUNK * GPT_A
    e_pad = ((e + unit - 1) // unit) * unit
    src_p = jnp.concatenate([src, jnp.full((e_pad - e,), n, jnp.int32)])
    dst_p = jnp.concatenate([dst, jnp.full((e_pad - e,), N_PAD - 1, jnp.int32)])
    emb_pad = jnp.pad(emb, ((0, N_PAD - n), (0, 0)))
    b1r = b1.reshape(1, d)
    b2r = b2.reshape(1, d)
    ones = jnp.ones((CHUNK, d), jnp.float32)

    deg_p = _sc_degree(dst_p, ones)
    y1 = _tc_in(emb_pad, deg_p, W1)
    src2d = src_p.reshape(-1, CHUNK)
    agg1 = _sc_edge_pass(y1, src2d, dst_p)
    y2 = _tc_mid(agg1, deg_p, b1r, W2)
    agg2 = _sc_edge_pass(y2, src2d, dst_p)
    out = _tc_out(agg2, deg_p, b2r)
    return out[:n]


# speedup vs baseline: 4.8476x; 4.8476x over previous
import functools

import jax
import jax.numpy as jnp
from jax import lax
from jax.experimental import pallas as pl
from jax.experimental.pallas import tpu as pltpu
from jax.experimental.pallas import tpu_sc as plsc

N_PAD = 10112
LANES = 16
CHUNK = 128
NTILES = 32
NBLOCKS = N_PAD // CHUNK
NBUF = 3


def _mesh():
    return plsc.VectorSubcoreMesh(core_axis_name="c", subcore_axis_name="s",
                                  num_cores=2, num_subcores=16)


def _tile_blocks(s):
    return [(k, s + k * 16) for k in range((NBLOCKS + 15) // 16)]


def _zero_vmem_block(buf_ref, d):
    def zbody(i, _):
        r = i // (d // LANES)
        col = (i % (d // LANES)) * LANES
        buf_ref[r, pl.ds(col, LANES)] = jnp.zeros((LANES,), jnp.float32)
        return 0
    lax.fori_loop(0, CHUNK * (d // LANES), zbody, 0)


def _sc_degree(dst1d, ones):
    g_per_tile = dst1d.shape[0] // (NTILES * CHUNK)
    d = ones.shape[1]

    @functools.partial(
        pl.kernel,
        out_type=jax.ShapeDtypeStruct((2, N_PAD, d), jnp.float32),
        mesh=_mesh(),
        scratch_types=[
            pltpu.VMEM((NBUF, CHUNK), jnp.int32),
            pltpu.VMEM((CHUNK, d), jnp.float32),
            pltpu.VMEM_SHARED((N_PAD, d), jnp.float32),
            [pltpu.SemaphoreType.DMA] * NBUF,
        ],
    )
    def k(dst_hbm, ones_hbm, out_hbm, ib_v, ones_v, deg_sh, isems):
        c = lax.axis_index("c")
        s = lax.axis_index("s")
        wid = c * 16 + s
        base = wid * g_per_tile

        _zero_vmem_block(ones_v, d)
        for _, blk in _tile_blocks(s):
            @pl.when(blk < NBLOCKS)
            def _():
                pltpu.sync_copy(ones_v, deg_sh.at[pl.ds(blk * CHUNK, CHUNK)])
        plsc.subcore_barrier()
        pltpu.sync_copy(ones_hbm, ones_v)

        def idx_load(g, b):
            pltpu.async_copy(dst_hbm.at[pl.ds((base + g) * CHUNK, CHUNK)],
                             ib_v.at[b], isems[b])

        for j in range(NBUF):
            idx_load(j, j)

        def step(t, _):
            for b in range(NBUF):
                g = t * NBUF + b
                pltpu.make_async_copy(dst_hbm.at[pl.ds((base + g) * CHUNK, CHUNK)],
                                      ib_v.at[b], isems[b]).wait()
                pltpu.sync_copy(ones_v, deg_sh.at[ib_v.at[b]], add=True)

                @pl.when(g + NBUF < g_per_tile)
                def _():
                    idx_load(g + NBUF, b)
            return 0
        lax.fori_loop(0, g_per_tile // NBUF, step, 0)

        plsc.subcore_barrier()
        for _, blk in _tile_blocks(s):
            @pl.when(blk < NBLOCKS)
            def _():
                pltpu.sync_copy(deg_sh.at[pl.ds(blk * CHUNK, CHUNK)],
                                out_hbm.at[c, pl.ds(blk * CHUNK, CHUNK)])

    return k(dst1d, ones)


def _sc_edge_pass(y, src1d, dst1d):
    g_per_tile = src1d.shape[0] // (NTILES * CHUNK)
    assert g_per_tile % NBUF == 0 and g_per_tile > 2 * NBUF
    d = y.shape[1]

    @functools.partial(
        pl.kernel,
        out_type=jax.ShapeDtypeStruct((2, N_PAD, d), jnp.float32),
        mesh=_mesh(),
        scratch_types=[
            pltpu.VMEM((NBUF, 2, CHUNK), jnp.int32),
            pltpu.VMEM((NBUF, CHUNK, d), jnp.float32),
            pltpu.VMEM_SHARED((N_PAD, d), jnp.float32),
            [pltpu.SemaphoreType.DMA] * NBUF,
            [pltpu.SemaphoreType.DMA] * NBUF,
            [pltpu.SemaphoreType.DMA] * NBUF,
        ],
    )
    def k(y_hbm, src_hbm, dst_hbm, out_hbm, ib_v, rows_v, agg_sh, gsems, isems,
          ssems):
        c = lax.axis_index("c")
        s = lax.axis_index("s")
        wid = c * 16 + s
        base = wid * g_per_tile

        _zero_vmem_block(rows_v.at[0], d)
        for _, blk in _tile_blocks(s):
            @pl.when(blk < NBLOCKS)
            def _():
                pltpu.sync_copy(rows_v.at[0], agg_sh.at[pl.ds(blk * CHUNK, CHUNK)])
        plsc.subcore_barrier()

        def idx_load(g, b):
            off = (base + g) * CHUNK
            pltpu.async_copy(src_hbm.at[pl.ds(off, CHUNK)], ib_v.at[b, 0], isems[b])
            pltpu.async_copy(dst_hbm.at[pl.ds(off, CHUNK)], ib_v.at[b, 1], isems[b])

        def idx_wait(g, b):
            off = (base + g) * CHUNK
            pltpu.make_async_copy(src_hbm.at[pl.ds(off, CHUNK)], ib_v.at[b, 0],
                                  isems[b]).wait()
            pltpu.make_async_copy(dst_hbm.at[pl.ds(off, CHUNK)], ib_v.at[b, 1],
                                  isems[b]).wait()

        def gather(g, b):
            pltpu.async_copy(y_hbm.at[ib_v.at[b, 0]], rows_v.at[b], gsems[b])

        def gather_wait(b):
            pltpu.make_async_copy(y_hbm.at[ib_v.at[b, 0]], rows_v.at[b],
                                  gsems[b]).wait()

        def scatter(b):
            pltpu.async_copy(rows_v.at[b], agg_sh.at[ib_v.at[b, 1]], ssems[b],
                             add=True)

        def scatter_wait(b):
            pltpu.make_async_copy(rows_v.at[b], agg_sh.at[ib_v.at[b, 1]],
                                  ssems[b]).wait()

        for j in range(NBUF):
            idx_load(j, j)
        for j in range(NBUF - 1):
            idx_wait(j, j)
            gather(j, j)

        gather_wait(0); scatter(0)
        idx_wait(2, 2); gather(2, 2)
        for g in (1, 2):
            b = g % NBUF
            bn = (g + 2) % NBUF
            gather_wait(b); scatter(b)
            scatter_wait(bn)
            idx_load(g + 2, bn)
            idx_wait(g + 2, bn)
            gather(g + 2, bn)

        def step(t, _):
            for j in range(NBUF):
                g = t * NBUF + j
                b = j
                bn = (j + 2) % NBUF
                gather_wait(b)
                scatter(b)
                scatter_wait(bn)
                @pl.when(g + 2 < g_per_tile)
                def _():
                    idx_load(g + 2, bn)
                    idx_wait(g + 2, bn)
                    gather(g + 2, bn)
            return 0
        lax.fori_loop(1, g_per_tile // NBUF, step, 0)
        scatter_wait((g_per_tile - 1) % NBUF)

        plsc.subcore_barrier()
        for _, blk in _tile_blocks(s):
            @pl.when(blk < NBLOCKS)
            def _():
                pltpu.sync_copy(agg_sh.at[pl.ds(blk * CHUNK, CHUNK)],
                                out_hbm.at[c, pl.ds(blk * CHUNK, CHUNK)])

    return k(y, src1d, dst1d)


def _dinv_block(deg_ref):
    deg = deg_ref[0, :, 0:1] + deg_ref[1, :, 0:1]
    return jnp.where(deg > 0, lax.rsqrt(deg), 0.0)


def _tc_in(x, deg_p, W):
    n, d = x.shape

    def body(x_ref, deg_ref, w_ref, y_ref):
        dinv = _dinv_block(deg_ref)
        y_ref[...] = dinv * jnp.dot(x_ref[...], w_ref[...],
                                    preferred_element_type=jnp.float32)

    return pl.pallas_call(
        body,
        grid=(n // 128,),
        in_specs=[
            pl.BlockSpec((128, d), lambda i: (i, 0)),
            pl.BlockSpec((2, 128, 128), lambda i: (0, i, 0)),
            pl.BlockSpec((d, d), lambda i: (0, 0)),
        ],
        out_specs=pl.BlockSpec((128, d), lambda i: (i, 0)),
        out_shape=jax.ShapeDtypeStruct((n, d), jnp.float32),
    )(x, deg_p, W)


def _tc_mid(agg_p, deg_p, b, W):
    n, d = agg_p.shape[1], agg_p.shape[2]

    def body(a_ref, deg_ref, b_ref, w_ref, y_ref):
        dinv = _dinv_block(deg_ref)
        x2 = jnp.maximum(dinv * (a_ref[0] + a_ref[1]) + b_ref[...], 0.0)
        y_ref[...] = dinv * jnp.dot(x2, w_ref[...], preferred_element_type=jnp.float32)

    return pl.pallas_call(
        body,
        grid=(n // 128,),
        in_specs=[
            pl.BlockSpec((2, 128, d), lambda i: (0, i, 0)),
            pl.BlockSpec((2, 128, 128), lambda i: (0, i, 0)),
            pl.BlockSpec((1, d), lambda i: (0, 0)),
            pl.BlockSpec((d, d), lambda i: (0, 0)),
        ],
        out_specs=pl.BlockSpec((128, d), lambda i: (i, 0)),
        out_shape=jax.ShapeDtypeStruct((n, d), jnp.float32),
    )(agg_p, deg_p, b, W)


def _tc_out(agg_p, deg_p, b):
    n, d = agg_p.shape[1], agg_p.shape[2]

    def body(a_ref, deg_ref, b_ref, o_ref):
        dinv = _dinv_block(deg_ref)
        o_ref[...] = jnp.maximum(dinv * (a_ref[0] + a_ref[1]) + b_ref[...], 0.0)

    return pl.pallas_call(
        body,
        grid=(n // 128,),
        in_specs=[
            pl.BlockSpec((2, 128, d), lambda i: (0, i, 0)),
            pl.BlockSpec((2, 128, 128), lambda i: (0, i, 0)),
            pl.BlockSpec((1, d), lambda i: (0, 0)),
        ],
        out_specs=pl.BlockSpec((128, d), lambda i: (i, 0)),
        out_shape=jax.ShapeDtypeStruct((n, d), jnp.float32),
    )(agg_p, deg_p, b)


def kernel(edge_index, emb, W1, b1, W2, b2):
    src, dst = edge_index[0], edge_index[1]
    e = src.shape[0]
    n, d = emb.shape

    unit = NTILES * CHUNK * NBUF
    e_pad = ((e + unit - 1) // unit) * unit
    src_p = jnp.concatenate([src, jnp.full((e_pad - e,), n, jnp.int32)])
    dst_p = jnp.concatenate([dst, jnp.full((e_pad - e,), N_PAD - 1, jnp.int32)])
    emb_pad = jnp.pad(emb, ((0, N_PAD - n), (0, 0)))
    b1r = b1.reshape(1, d)
    b2r = b2.reshape(1, d)
    ones = jnp.ones((CHUNK, d), jnp.float32)

    deg_p = _sc_degree(dst_p, ones)
    y1 = _tc_in(emb_pad, deg_p, W1)
    agg1 = _sc_edge_pass(y1, src_p, dst_p)
    y2 = _tc_mid(agg1, deg_p, b1r, W2)
    agg2 = _sc_edge_pass(y2, src_p, dst_p)
    out = _tc_out(agg2, deg_p, b2r)
    return out[:n]


# speedup vs baseline: 5.6749x; 1.1707x over previous
import functools

import jax
import jax.numpy as jnp
from jax import lax
from jax.experimental import pallas as pl
from jax.experimental.pallas import tpu as pltpu
from jax.experimental.pallas import tpu_sc as plsc

N_PAD = 10112
LANES = 16
CHUNK = 128
NTILES = 32
NBLOCKS = N_PAD // CHUNK
NBUF = 3


def _mesh():
    return plsc.VectorSubcoreMesh(core_axis_name="c", subcore_axis_name="s",
                                  num_cores=2, num_subcores=16)


def _tile_blocks(s):
    return [(k, s + k * 16) for k in range((NBLOCKS + 15) // 16)]


def _zero_vmem_block(buf_ref, d):
    def zbody(i, _):
        r = i // (d // LANES)
        col = (i % (d // LANES)) * LANES
        buf_ref[r, pl.ds(col, LANES)] = jnp.zeros((LANES,), jnp.float32)
        return 0
    lax.fori_loop(0, CHUNK * (d // LANES), zbody, 0)


def _sc_degree(dst1d, ones):
    g_per_tile = dst1d.shape[0] // (NTILES * CHUNK)
    d = ones.shape[1]
    R = 4
    assert g_per_tile % R == 0

    @functools.partial(
        pl.kernel,
        out_type=jax.ShapeDtypeStruct((2, N_PAD, d), jnp.float32),
        mesh=_mesh(),
        scratch_types=[
            pltpu.VMEM((R, CHUNK), jnp.int32),
            pltpu.VMEM((CHUNK, d), jnp.float32),
            pltpu.VMEM_SHARED((N_PAD, d), jnp.float32),
            [pltpu.SemaphoreType.DMA] * R,
        ],
    )
    def k(dst_hbm, ones_hbm, out_hbm, ib_v, ones_v, deg_sh, isems):
        c = lax.axis_index("c")
        s = lax.axis_index("s")
        wid = c * 16 + s
        base = wid * g_per_tile

        _zero_vmem_block(ones_v, d)
        for _, blk in _tile_blocks(s):
            @pl.when(blk < NBLOCKS)
            def _():
                pltpu.sync_copy(ones_v, deg_sh.at[pl.ds(blk * CHUNK, CHUNK)])
        plsc.subcore_barrier()
        pltpu.sync_copy(ones_hbm, ones_v)

        def idx_load(g, b):
            pltpu.async_copy(dst_hbm.at[pl.ds((base + g) * CHUNK, CHUNK)],
                             ib_v.at[b], isems[b])

        for j in range(R):
            idx_load(j, j)

        def step(t, _):
            for b in range(R):
                g = t * R + b
                pltpu.make_async_copy(dst_hbm.at[pl.ds((base + g) * CHUNK, CHUNK)],
                                      ib_v.at[b], isems[b]).wait()
                pltpu.sync_copy(ones_v, deg_sh.at[ib_v.at[b]], add=True)

                @pl.when(g + R < g_per_tile)
                def _():
                    idx_load(g + R, b)
            return 0
        lax.fori_loop(0, g_per_tile // R, step, 0)

        plsc.subcore_barrier()
        for _, blk in _tile_blocks(s):
            @pl.when(blk < NBLOCKS)
            def _():
                pltpu.sync_copy(deg_sh.at[pl.ds(blk * CHUNK, CHUNK)],
                                out_hbm.at[c, pl.ds(blk * CHUNK, CHUNK)])

    return k(dst1d, ones)


def _sc_edge_pass(y, src2d, dst1d):
    g_per_tile = src2d.shape[0] // NTILES
    assert g_per_tile % 4 == 0 and g_per_tile % 8 == 0
    d = y.shape[1]

    @functools.partial(
        pl.kernel,
        out_type=jax.ShapeDtypeStruct((2, N_PAD, d), jnp.float32),
        mesh=_mesh(),
        scratch_types=[
            pltpu.VMEM((g_per_tile, CHUNK), jnp.int32),
            pltpu.VMEM((4, CHUNK), jnp.int32),
            pltpu.VMEM((2, CHUNK, d), jnp.float32),
            pltpu.VMEM_SHARED((N_PAD, d), jnp.float32),
            [pltpu.SemaphoreType.DMA] * 2,
            [pltpu.SemaphoreType.DMA] * 4,
            [pltpu.SemaphoreType.DMA] * 2,
        ],
    )
    def k(y_hbm, src_hbm, dst_hbm, out_hbm, src_v, dstb_v, rows_v, agg_sh,
          gsems, isems, ssems):
        c = lax.axis_index("c")
        s = lax.axis_index("s")
        wid = c * 16 + s
        base = wid * g_per_tile

        _zero_vmem_block(rows_v.at[0], d)
        for _, blk in _tile_blocks(s):
            @pl.when(blk < NBLOCKS)
            def _():
                pltpu.sync_copy(rows_v.at[0], agg_sh.at[pl.ds(blk * CHUNK, CHUNK)])
        plsc.subcore_barrier()

        pltpu.sync_copy(src_hbm.at[pl.ds(base, g_per_tile)], src_v)

        def dst_load(g, b4):
            pltpu.async_copy(dst_hbm.at[pl.ds((base + g) * CHUNK, CHUNK)],
                             dstb_v.at[b4], isems[b4])

        def dst_wait(g, b4):
            pltpu.make_async_copy(dst_hbm.at[pl.ds((base + g) * CHUNK, CHUNK)],
                                  dstb_v.at[b4], isems[b4]).wait()

        def gather(g, b2):
            pltpu.async_copy(y_hbm.at[src_v.at[g]], rows_v.at[b2], gsems[b2])

        def gather_wait(g, b2):
            pltpu.make_async_copy(y_hbm.at[src_v.at[g]], rows_v.at[b2],
                                  gsems[b2]).wait()

        def scatter(b2, b4):
            pltpu.async_copy(rows_v.at[b2], agg_sh.at[dstb_v.at[b4]], ssems[b2],
                             add=True)

        def scatter_wait(b2, b4):
            pltpu.make_async_copy(rows_v.at[b2], agg_sh.at[dstb_v.at[b4]],
                                  ssems[b2]).wait()

        for j in range(4):
            dst_load(j, j)
        gather(0, 0)
        gather(1, 1)
        gather_wait(0, 0)
        dst_wait(0, 0)
        scatter(0, 0)

        def body(g, b2, b4):
            scatter_wait(1 - b2, (b4 + 3) % 4)
            @pl.when(g + 3 < g_per_tile)
            def _():
                dst_load(g + 3, (b4 + 3) % 4)
            @pl.when(g + 1 < g_per_tile)
            def _():
                gather(g + 1, 1 - b2)
            gather_wait(g, b2)
            dst_wait(g, b4)
            scatter(b2, b4)

        for g0 in (1, 2, 3):
            body(g0, g0 % 2, g0 % 4)

        def step(t, _):
            for j in range(4):
                body(t * 4 + j, j % 2, j)
            return 0
        lax.fori_loop(1, g_per_tile // 4, step, 0)
        scatter_wait((g_per_tile - 1) % 2, 3)

        plsc.subcore_barrier()
        for _, blk in _tile_blocks(s):
            @pl.when(blk < NBLOCKS)
            def _():
                pltpu.sync_copy(agg_sh.at[pl.ds(blk * CHUNK, CHUNK)],
                                out_hbm.at[c, pl.ds(blk * CHUNK, CHUNK)])

    return k(y, src2d, dst1d)


def _dinv_block(deg_ref):
    deg = deg_ref[0, :, 0:1] + deg_ref[1, :, 0:1]
    return jnp.where(deg > 0, lax.rsqrt(deg), 0.0)


def _tc_in(x, deg_p, W):
    n, d = x.shape

    def body(x_ref, deg_ref, w_ref, y_ref):
        dinv = _dinv_block(deg_ref)
        y_ref[...] = dinv * jnp.dot(x_ref[...], w_ref[...],
                                    preferred_element_type=jnp.float32)

    return pl.pallas_call(
        body,
        grid=(n // 128,),
        in_specs=[
            pl.BlockSpec((128, d), lambda i: (i, 0)),
            pl.BlockSpec((2, 128, 128), lambda i: (0, i, 0)),
            pl.BlockSpec((d, d), lambda i: (0, 0)),
        ],
        out_specs=pl.BlockSpec((128, d), lambda i: (i, 0)),
        out_shape=jax.ShapeDtypeStruct((n, d), jnp.float32),
    )(x, deg_p, W)


def _tc_mid(agg_p, deg_p, b, W):
    n, d = agg_p.shape[1], agg_p.shape[2]

    def body(a_ref, deg_ref, b_ref, w_ref, y_ref):
        dinv = _dinv_block(deg_ref)
        x2 = jnp.maximum(dinv * (a_ref[0] + a_ref[1]) + b_ref[...], 0.0)
        y_ref[...] = dinv * jnp.dot(x2, w_ref[...], preferred_element_type=jnp.float32)

    return pl.pallas_call(
        body,
        grid=(n // 128,),
        in_specs=[
            pl.BlockSpec((2, 128, d), lambda i: (0, i, 0)),
            pl.BlockSpec((2, 128, 128), lambda i: (0, i, 0)),
            pl.BlockSpec((1, d), lambda i: (0, 0)),
            pl.BlockSpec((d, d), lambda i: (0, 0)),
        ],
        out_specs=pl.BlockSpec((128, d), lambda i: (i, 0)),
        out_shape=jax.ShapeDtypeStruct((n, d), jnp.float32),
    )(agg_p, deg_p, b, W)


def _tc_out(agg_p, deg_p, b):
    n, d = agg_p.shape[1], agg_p.shape[2]

    def body(a_ref, deg_ref, b_ref, o_ref):
        dinv = _dinv_block(deg_ref)
        o_ref[...] = jnp.maximum(dinv * (a_ref[0] + a_ref[1]) + b_ref[...], 0.0)

    return pl.pallas_call(
        body,
        grid=(n // 128,),
        in_specs=[
            pl.BlockSpec((2, 128, d), lambda i: (0, i, 0)),
            pl.BlockSpec((2, 128, 128), lambda i: (0, i, 0)),
            pl.BlockSpec((1, d), lambda i: (0, 0)),
        ],
        out_specs=pl.BlockSpec((128, d), lambda i: (i, 0)),
        out_shape=jax.ShapeDtypeStruct((n, d), jnp.float32),
    )(agg_p, deg_p, b)


def kernel(edge_index, emb, W1, b1, W2, b2):
    src, dst = edge_index[0], edge_index[1]
    e = src.shape[0]
    n, d = emb.shape

    unit = NTILES * CHUNK * 8
    e_pad = ((e + unit - 1) // unit) * unit
    src_p = jnp.concatenate([src, jnp.full((e_pad - e,), n, jnp.int32)])
    dst_p = jnp.concatenate([dst, jnp.full((e_pad - e,), N_PAD - 1, jnp.int32)])
    emb_pad = jnp.pad(emb, ((0, N_PAD - n), (0, 0)))
    b1r = b1.reshape(1, d)
    b2r = b2.reshape(1, d)
    ones = jnp.ones((CHUNK, d), jnp.float32)

    deg_p = _sc_degree(dst_p, ones)
    y1 = _tc_in(emb_pad, deg_p, W1)
    src2d = src_p.reshape(-1, CHUNK)
    agg1 = _sc_edge_pass(y1, src2d, dst_p)
    y2 = _tc_mid(agg1, deg_p, b1r, W2)
    agg2 = _sc_edge_pass(y2, src2d, dst_p)
    out = _tc_out(agg2, deg_p, b2r)
    return out[:n]


# speedup vs baseline: 5.6785x; 1.0006x over previous
import functools

import jax
import jax.numpy as jnp
from jax import lax
from jax.experimental import pallas as pl
from jax.experimental.pallas import tpu as pltpu
from jax.experimental.pallas import tpu_sc as plsc

N_PAD = 10112
LANES = 16
CHUNK = 128
NTILES = 32
NBLOCKS = N_PAD // CHUNK


def _mesh():
    return plsc.VectorSubcoreMesh(core_axis_name="c", subcore_axis_name="s",
                                  num_cores=2, num_subcores=16)


def _tile_blocks(s):
    return [(k, s + k * 16) for k in range((NBLOCKS + 15) // 16)]


def _zero_vmem_block(buf_ref, d):
    def zbody(i, _):
        r = i // (d // LANES)
        col = (i % (d // LANES)) * LANES
        buf_ref[r, pl.ds(col, LANES)] = jnp.zeros((LANES,), jnp.float32)
        return 0
    lax.fori_loop(0, CHUNK * (d // LANES), zbody, 0)


def _sc_degree(dst1d, ones):
    g_per_tile = dst1d.shape[0] // (NTILES * CHUNK)
    d = ones.shape[1]
    R = 4
    assert g_per_tile % R == 0

    @functools.partial(
        pl.kernel,
        out_type=jax.ShapeDtypeStruct((2, N_PAD, d), jnp.float32),
        mesh=_mesh(),
        scratch_types=[
            pltpu.VMEM((R, CHUNK), jnp.int32),
            pltpu.VMEM((CHUNK, d), jnp.float32),
            pltpu.VMEM_SHARED((N_PAD, d), jnp.float32),
            [pltpu.SemaphoreType.DMA] * R,
        ],
    )
    def k(dst_hbm, ones_hbm, out_hbm, ib_v, ones_v, deg_sh, isems):
        c = lax.axis_index("c")
        s = lax.axis_index("s")
        wid = c * 16 + s
        base = wid * g_per_tile

        _zero_vmem_block(ones_v, d)
        for _, blk in _tile_blocks(s):
            @pl.when(blk < NBLOCKS)
            def _():
                pltpu.sync_copy(ones_v, deg_sh.at[pl.ds(blk * CHUNK, CHUNK)])
        plsc.subcore_barrier()
        pltpu.sync_copy(ones_hbm, ones_v)

        def idx_load(g, b):
            pltpu.async_copy(dst_hbm.at[pl.ds((base + g) * CHUNK, CHUNK)],
                             ib_v.at[b], isems[b])

        for j in range(R):
            idx_load(j, j)

        def step(t, _):
            for b in range(R):
                g = t * R + b
                pltpu.make_async_copy(dst_hbm.at[pl.ds((base + g) * CHUNK, CHUNK)],
                                      ib_v.at[b], isems[b]).wait()
                pltpu.sync_copy(ones_v, deg_sh.at[ib_v.at[b]], add=True)

                @pl.when(g + R < g_per_tile)
                def _():
                    idx_load(g + R, b)
            return 0
        lax.fori_loop(0, g_per_tile // R, step, 0)

        plsc.subcore_barrier()
        for _, blk in _tile_blocks(s):
            @pl.when(blk < NBLOCKS)
            def _():
                pltpu.sync_copy(deg_sh.at[pl.ds(blk * CHUNK, CHUNK)],
                                out_hbm.at[c, pl.ds(blk * CHUNK, CHUNK)])

    return k(dst1d, ones)


def _sc_edge_pass(y, src2d, dst1d):
    g_per_tile = src2d.shape[0] // NTILES
    assert g_per_tile % 4 == 0 and g_per_tile % 8 == 0
    d = y.shape[1]

    @functools.partial(
        pl.kernel,
        out_type=jax.ShapeDtypeStruct((2, N_PAD, d), jnp.float32),
        mesh=_mesh(),
        scratch_types=[
            pltpu.VMEM((g_per_tile, CHUNK), jnp.int32),
            pltpu.VMEM((4, CHUNK), jnp.int32),
            pltpu.VMEM((2, CHUNK, d), jnp.float32),
            pltpu.VMEM_SHARED((N_PAD, d), jnp.float32),
            [pltpu.SemaphoreType.DMA] * 2,
            [pltpu.SemaphoreType.DMA] * 4,
            [pltpu.SemaphoreType.DMA] * 2,
        ],
    )
    def k(y_hbm, src_hbm, dst_hbm, out_hbm, src_v, dstb_v, rows_v, agg_sh,
          gsems, isems, ssems):
        c = lax.axis_index("c")
        s = lax.axis_index("s")
        wid = c * 16 + s
        base = wid * g_per_tile

        _zero_vmem_block(rows_v.at[0], d)
        for _, blk in _tile_blocks(s):
            @pl.when(blk < NBLOCKS)
            def _():
                pltpu.sync_copy(rows_v.at[0], agg_sh.at[pl.ds(blk * CHUNK, CHUNK)])
        plsc.subcore_barrier()

        pltpu.sync_copy(src_hbm.at[pl.ds(base, g_per_tile)], src_v)

        def dst_load(g, b4):
            pltpu.async_copy(dst_hbm.at[pl.ds((base + g) * CHUNK, CHUNK)],
                             dstb_v.at[b4], isems[b4])

        def dst_wait(g, b4):
            pltpu.make_async_copy(dst_hbm.at[pl.ds((base + g) * CHUNK, CHUNK)],
                                  dstb_v.at[b4], isems[b4]).wait()

        def gather(g, b2):
            pltpu.async_copy(y_hbm.at[src_v.at[g]], rows_v.at[b2], gsems[b2])

        def gather_wait(g, b2):
            pltpu.make_async_copy(y_hbm.at[src_v.at[g]], rows_v.at[b2],
                                  gsems[b2]).wait()

        def scatter(b2, b4):
            pltpu.async_copy(rows_v.at[b2], agg_sh.at[dstb_v.at[b4]], ssems[b2],
                             add=True)

        def scatter_wait(b2, b4):
            pltpu.make_async_copy(rows_v.at[b2], agg_sh.at[dstb_v.at[b4]],
                                  ssems[b2]).wait()

        for j in range(4):
            dst_load(j, j)
        gather(0, 0)
        gather(1, 1)
        gather_wait(0, 0)
        dst_wait(0, 0)
        scatter(0, 0)

        def body(g, b2, b4):
            scatter_wait(1 - b2, (b4 + 3) % 4)
            @pl.when(g + 3 < g_per_tile)
            def _():
                dst_load(g + 3, (b4 + 3) % 4)
            @pl.when(g + 1 < g_per_tile)
            def _():
                gather(g + 1, 1 - b2)
            gather_wait(g, b2)
            dst_wait(g, b4)
            scatter(b2, b4)

        for g0 in (1, 2, 3):
            body(g0, g0 % 2, g0 % 4)

        def step(t, _):
            for j in range(4):
                body(t * 4 + j, j % 2, j)
            return 0
        lax.fori_loop(1, g_per_tile // 4, step, 0)
        scatter_wait((g_per_tile - 1) % 2, 3)

        plsc.subcore_barrier()
        for _, blk in _tile_blocks(s):
            @pl.when(blk < NBLOCKS)
            def _():
                pltpu.sync_copy(agg_sh.at[pl.ds(blk * CHUNK, CHUNK)],
                                out_hbm.at[c, pl.ds(blk * CHUNK, CHUNK)])

    return k(y, src2d, dst1d)


def _dinv_block(deg_ref):
    deg = deg_ref[0, :, 0:1] + deg_ref[1, :, 0:1]
    return jnp.where(deg > 0, lax.rsqrt(deg), 0.0)


def _tc_in(x, deg_p, W):
    n, d = x.shape

    def body(x_ref, deg_ref, w_ref, y_ref):
        dinv = _dinv_block(deg_ref)
        y_ref[...] = dinv * jnp.dot(x_ref[...], w_ref[...],
                                    preferred_element_type=jnp.float32)

    return pl.pallas_call(
        body,
        grid=(n // 128,),
        in_specs=[
            pl.BlockSpec((128, d), lambda i: (i, 0)),
            pl.BlockSpec((2, 128, 128), lambda i: (0, i, 0)),
            pl.BlockSpec((d, d), lambda i: (0, 0)),
        ],
        out_specs=pl.BlockSpec((128, d), lambda i: (i, 0)),
        out_shape=jax.ShapeDtypeStruct((n, d), jnp.float32),
    )(x, deg_p, W)


def _tc_mid(agg_p, deg_p, b, W):
    n, d = agg_p.shape[1], agg_p.shape[2]

    def body(a_ref, deg_ref, b_ref, w_ref, y_ref):
        dinv = _dinv_block(deg_ref)
        x2 = jnp.maximum(dinv * (a_ref[0] + a_ref[1]) + b_ref[...], 0.0)
        y_ref[...] = dinv * jnp.dot(x2, w_ref[...], preferred_element_type=jnp.float32)

    return pl.pallas_call(
        body,
        grid=(n // 128,),
        in_specs=[
            pl.BlockSpec((2, 128, d), lambda i: (0, i, 0)),
            pl.BlockSpec((2, 128, 128), lambda i: (0, i, 0)),
            pl.BlockSpec((1, d), lambda i: (0, 0)),
            pl.BlockSpec((d, d), lambda i: (0, 0)),
        ],
        out_specs=pl.BlockSpec((128, d), lambda i: (i, 0)),
        out_shape=jax.ShapeDtypeStruct((n, d), jnp.float32),
    )(agg_p, deg_p, b, W)


def _tc_out(agg_p, deg_p, b):
    n, d = agg_p.shape[1], agg_p.shape[2]

    def body(a_ref, deg_ref, b_ref, o_ref):
        dinv = _dinv_block(deg_ref)
        o_ref[...] = jnp.maximum(dinv * (a_ref[0] + a_ref[1]) + b_ref[...], 0.0)

    return pl.pallas_call(
        body,
        grid=(n // 128,),
        in_specs=[
            pl.BlockSpec((2, 128, d), lambda i: (0, i, 0)),
            pl.BlockSpec((2, 128, 128), lambda i: (0, i, 0)),
            pl.BlockSpec((1, d), lambda i: (0, 0)),
        ],
        out_specs=pl.BlockSpec((128, d), lambda i: (i, 0)),
        out_shape=jax.ShapeDtypeStruct((n, d), jnp.float32),
    )(agg_p, deg_p, b)


def kernel(edge_index, emb, W1, b1, W2, b2):
    src, dst = edge_index[0], edge_index[1]
    e = src.shape[0]
    n, d = emb.shape

    unit = NTILES * CHUNK * 8
    e_pad = ((e + unit - 1) // unit) * unit
    src_p = jnp.concatenate([src, jnp.full((e_pad - e,), n, jnp.int32)])
    dst_p = jnp.concatenate([dst, jnp.full((e_pad - e,), N_PAD - 1, jnp.int32)])
    emb_pad = jnp.pad(emb, ((0, N_PAD - n), (0, 0)))
    b1r = b1.reshape(1, d)
    b2r = b2.reshape(1, d)
    ones = jnp.ones((CHUNK, d), jnp.float32)

    deg_p = _sc_degree(dst_p, ones)
    y1 = _tc_in(emb_pad, deg_p, W1)
    src2d = src_p.reshape(-1, CHUNK)
    agg1 = _sc_edge_pass(y1, src2d, dst_p)
    y2 = _tc_mid(agg1, deg_p, b1r, W2)
    agg2 = _sc_edge_pass(y2, src2d, dst_p)
    out = _tc_out(agg2, deg_p, b2r)
    return out[:n]
